# Initial kernel scaffold; baseline (speedup 1.0000x reference)
import jax
import jax.numpy as jnp
from jax.experimental import pallas as pl


def kernel(x_drug, x_protein, edge_dd, edge_dt, edge_td, edge_pp, Wp_drug, bp_drug, Wp_protein, bp_protein, W_rel, W_self, ln_gamma, ln_beta):
    raise NotImplementedError("write your pallas kernel here")


# speedup vs baseline: 6.2012x; 6.2012x over previous
import functools

import jax
import jax.numpy as jnp
from jax import lax
from jax.experimental import pallas as pl
from jax.experimental.pallas import tpu as pltpu
from jax.experimental.pallas import tpu_sc as plsc

ND = 10000
D = 128
DIN = 256
E = 131072
NC = 2
NS = 16
EROWS = E // 128
RPT = EROWS // (NC * NS)
NPT = ND // NS


def _proj_body(x_ref, w_ref, b_ref, o_ref):
    o_ref[...] = jnp.maximum(
        jnp.dot(x_ref[...], w_ref[...], preferred_element_type=jnp.float32)
        + b_ref[...], 0.0)


def _input_proj(x, w_t, b):
    blk = 2000
    return pl.pallas_call(
        _proj_body,
        grid=(ND // blk,),
        in_specs=[pl.BlockSpec((blk, DIN), lambda i: (i, 0)),
                  pl.BlockSpec((DIN, D), lambda i: (0, 0)),
                  pl.BlockSpec((1, D), lambda i: (0, 0))],
        out_specs=pl.BlockSpec((blk, D), lambda i: (i, 0)),
        out_shape=jax.ShapeDtypeStruct((ND, D), jnp.float32),
    )(x, w_t, b)


def _msg_body(h_ref, w_ref, oa_ref, ob_ref, os_ref):
    r = jnp.dot(h_ref[...], w_ref[...], preferred_element_type=jnp.float32)
    oa_ref[...] = r[:, :D]
    ob_ref[...] = r[:, D:2 * D]
    os_ref[...] = r[:, 2 * D:]


def _msg(h, w_cat):
    blk = 2000
    return pl.pallas_call(
        _msg_body,
        grid=(ND // blk,),
        in_specs=[pl.BlockSpec((blk, D), lambda i: (i, 0)),
                  pl.BlockSpec((D, 3 * D), lambda i: (0, 0))],
        out_specs=[pl.BlockSpec((blk, D), lambda i: (i, 0))] * 3,
        out_shape=[jax.ShapeDtypeStruct((ND, D), jnp.float32)] * 3,
    )(h, w_cat)


def _combine_body(h_ref, s_ref, agg_ref, cnt_ref, g_ref, b_ref, o_ref):
    agg = agg_ref[0] + agg_ref[1]
    cnt = jnp.sum(cnt_ref[...], axis=1, keepdims=True)
    t = jnp.maximum(s_ref[...] + agg / jnp.maximum(cnt, 1.0), 0.0) + h_ref[...]
    mu = jnp.mean(t, axis=-1, keepdims=True)
    var = jnp.mean((t - mu) ** 2, axis=-1, keepdims=True)
    o_ref[...] = (t - mu) * lax.rsqrt(var + 1e-5) * g_ref[...] + b_ref[...]


def _combine(h, s, agg, cnt, g, b):
    blk = 2000
    return pl.pallas_call(
        _combine_body,
        grid=(ND // blk,),
        in_specs=[pl.BlockSpec((blk, D), lambda i: (i, 0)),
                  pl.BlockSpec((blk, D), lambda i: (i, 0)),
                  pl.BlockSpec((2, blk, D), lambda i: (0, i, 0)),
                  pl.BlockSpec((blk, NC * NS), lambda i: (i, 0)),
                  pl.BlockSpec((1, D), lambda i: (0, 0)),
                  pl.BlockSpec((1, D), lambda i: (0, 0))],
        out_specs=pl.BlockSpec((blk, D), lambda i: (i, 0)),
        out_shape=jax.ShapeDtypeStruct((ND, D), jnp.float32),
    )(h, s, agg, cnt, g, b)


def _sc_body(m_dd, m_dt, m_td, m_pp,
             s_dd, d_dd, s_dt, d_dt, s_td, d_td, s_pp, d_pp, zrows,
             aggd, aggp, cntd, cntp,
             acc, src_v, dst_v, rows_v, cnt_d, cnt_p, sem):
    cid = lax.axis_index("c")
    sid = lax.axis_index("s")
    wid = cid * NS + sid

    zero16 = jnp.zeros((16,), jnp.float32)
    one16 = jnp.ones((16,), jnp.float32)

    def _zcnt(i, carry):
        cnt_d[pl.ds(i * 16, 16)] = zero16
        cnt_p[pl.ds(i * 16, 16)] = zero16
        return carry
    lax.fori_loop(0, ND // 16, _zcnt, 0)

    def zero_acc():
        def _z(k, carry):
            pltpu.sync_copy(zrows, acc.at[pl.ds(sid * NPT + k * 125, 125)])
            return carry
        lax.fori_loop(0, NPT // 125, _z, 0)
        plsc.subcore_barrier()

    def do_rel(msg, s2d, d2d, cnt_loc):
        row0 = cid * (EROWS // 2) + sid * RPT
        pltpu.sync_copy(s2d.at[pl.ds(row0, RPT)], src_v)
        pltpu.sync_copy(d2d.at[pl.ds(row0, RPT)], dst_v)

        def _chunk(ci, carry):
            pltpu.async_copy(msg.at[src_v.at[ci]], rows_v, sem).wait()
            pltpu.sync_copy(rows_v, acc.at[dst_v.at[ci]], add=True)

            def _cl(j, c2):
                idx = dst_v[ci, pl.ds(j * 16, 16)]
                plsc.addupdate_scatter(cnt_loc, [idx], one16)
                return c2
            lax.fori_loop(0, 8, _cl, 0)
            return carry
        lax.fori_loop(0, RPT, _chunk, 0)

    def dump(out):
        plsc.subcore_barrier()
        pltpu.sync_copy(acc.at[pl.ds(sid * NPT, NPT)],
                        out.at[cid].at[pl.ds(sid * NPT, NPT)])
        plsc.subcore_barrier()

    zero_acc()
    do_rel(m_dd, s_dd, d_dd, cnt_d)
    do_rel(m_td, s_td, d_td, cnt_d)
    dump(aggd)
    zero_acc()
    do_rel(m_dt, s_dt, d_dt, cnt_p)
    do_rel(m_pp, s_pp, d_pp, cnt_p)
    dump(aggp)
    pltpu.sync_copy(cnt_d, cntd.at[wid])
    pltpu.sync_copy(cnt_p, cntp.at[wid])


_sc_agg = functools.partial(
    pl.kernel,
    out_type=[jax.ShapeDtypeStruct((NC, ND, D), jnp.float32),
              jax.ShapeDtypeStruct((NC, ND, D), jnp.float32),
              jax.ShapeDtypeStruct((NC * NS, ND), jnp.float32),
              jax.ShapeDtypeStruct((NC * NS, ND), jnp.float32)],
    mesh=plsc.VectorSubcoreMesh(core_axis_name="c", subcore_axis_name="s",
                                num_cores=NC, num_subcores=NS),
    compiler_params=pltpu.CompilerParams(use_tc_tiling_on_sc=False,
                                         needs_layout_passes=False),
    scratch_types=[
        pltpu.VMEM_SHARED((ND, D), jnp.float32),
        pltpu.VMEM((RPT, 128), jnp.int32),
        pltpu.VMEM((RPT, 128), jnp.int32),
        pltpu.VMEM((128, D), jnp.float32),
        pltpu.VMEM((ND,), jnp.float32),
        pltpu.VMEM((ND,), jnp.float32),
        pltpu.SemaphoreType.DMA,
    ],
)(_sc_body)


def kernel(x_drug, x_protein, edge_dd, edge_dt, edge_td, edge_pp,
           Wp_drug, bp_drug, Wp_protein, bp_protein,
           W_rel, W_self, ln_gamma, ln_beta):
    h_d = _input_proj(x_drug, Wp_drug.T, bp_drug[None, :])
    h_p = _input_proj(x_protein, Wp_protein.T, bp_protein[None, :])

    def split(e):
        e = e.astype(jnp.int32)
        return e[0].reshape(EROWS, 128), e[1].reshape(EROWS, 128)

    s_dd, d_dd = split(edge_dd)
    s_dt, d_dt = split(edge_dt)
    s_td, d_td = split(edge_td)
    s_pp, d_pp = split(edge_pp)
    zrows = jnp.zeros((125, D), jnp.float32)

    for l in range(2):
        w_d = jnp.concatenate(
            [W_rel[l, 0].T, W_rel[l, 1].T, W_self[l, 0].T], axis=1)
        w_p = jnp.concatenate(
            [W_rel[l, 2].T, W_rel[l, 3].T, W_self[l, 1].T], axis=1)
        m_dd, m_dt, self_d = _msg(h_d, w_d)
        m_td, m_pp, self_p = _msg(h_p, w_p)
        aggd, aggp, cntd, cntp = _sc_agg(
            m_dd, m_dt, m_td, m_pp,
            s_dd, d_dd, s_dt, d_dt, s_td, d_td, s_pp, d_pp, zrows)
        h_d = _combine(h_d, self_d, aggd, cntd.T,
                       ln_gamma[l, 0][None, :], ln_beta[l, 0][None, :])
        h_p = _combine(h_p, self_p, aggp, cntp.T,
                       ln_gamma[l, 1][None, :], ln_beta[l, 1][None, :])

    return jnp.concatenate([h_d, h_p], axis=0)


# speedup vs baseline: 7.6127x; 1.2276x over previous
import functools

import jax
import jax.numpy as jnp
from jax import lax
from jax.experimental import pallas as pl
from jax.experimental.pallas import tpu as pltpu
from jax.experimental.pallas import tpu_sc as plsc

ND = 10000
D = 128
DIN = 256
E = 131072
NC = 2
NS = 16
EROWS = E // 128
RPT = EROWS // (NC * NS)
NPT = ND // NS


def _proj_body(x_ref, w_ref, b_ref, o_ref):
    o_ref[...] = jnp.maximum(
        jnp.dot(x_ref[...], w_ref[...], preferred_element_type=jnp.float32)
        + b_ref[...], 0.0)


def _input_proj(x, w_t, b):
    blk = 2000
    return pl.pallas_call(
        _proj_body,
        grid=(ND // blk,),
        in_specs=[pl.BlockSpec((blk, DIN), lambda i: (i, 0)),
                  pl.BlockSpec((DIN, D), lambda i: (0, 0)),
                  pl.BlockSpec((1, D), lambda i: (0, 0))],
        out_specs=pl.BlockSpec((blk, D), lambda i: (i, 0)),
        out_shape=jax.ShapeDtypeStruct((ND, D), jnp.float32),
    )(x, w_t, b)


def _msg_body(h_ref, w_ref, oa_ref, ob_ref, os_ref):
    r = jnp.dot(h_ref[...], w_ref[...], preferred_element_type=jnp.float32)
    oa_ref[...] = r[:, :D]
    ob_ref[...] = r[:, D:2 * D]
    os_ref[...] = r[:, 2 * D:]


def _msg(h, w_cat):
    blk = 2000
    return pl.pallas_call(
        _msg_body,
        grid=(ND // blk,),
        in_specs=[pl.BlockSpec((blk, D), lambda i: (i, 0)),
                  pl.BlockSpec((D, 3 * D), lambda i: (0, 0))],
        out_specs=[pl.BlockSpec((blk, D), lambda i: (i, 0))] * 3,
        out_shape=[jax.ShapeDtypeStruct((ND, D), jnp.float32)] * 3,
    )(h, w_cat)


def _combine_body(h_ref, s_ref, agg_ref, cnt_ref, g_ref, b_ref, o_ref):
    agg = agg_ref[0] + agg_ref[1]
    cnt = jnp.sum(cnt_ref[...], axis=1, keepdims=True)
    t = jnp.maximum(s_ref[...] + agg / jnp.maximum(cnt, 1.0), 0.0) + h_ref[...]
    mu = jnp.mean(t, axis=-1, keepdims=True)
    var = jnp.mean((t - mu) ** 2, axis=-1, keepdims=True)
    o_ref[...] = (t - mu) * lax.rsqrt(var + 1e-5) * g_ref[...] + b_ref[...]


def _combine(h, s, agg, cnt, g, b):
    blk = 2000
    return pl.pallas_call(
        _combine_body,
        grid=(ND // blk,),
        in_specs=[pl.BlockSpec((blk, D), lambda i: (i, 0)),
                  pl.BlockSpec((blk, D), lambda i: (i, 0)),
                  pl.BlockSpec((2, blk, D), lambda i: (0, i, 0)),
                  pl.BlockSpec((blk, NC * NS), lambda i: (i, 0)),
                  pl.BlockSpec((1, D), lambda i: (0, 0)),
                  pl.BlockSpec((1, D), lambda i: (0, 0))],
        out_specs=pl.BlockSpec((blk, D), lambda i: (i, 0)),
        out_shape=jax.ShapeDtypeStruct((ND, D), jnp.float32),
    )(h, s, agg, cnt, g, b)


def _sc_body(m_dd, m_dt, m_td, m_pp,
             s_dd, d_dd, s_dt, d_dt, s_td, d_td, s_pp, d_pp, zrows,
             aggd, aggp, cntd, cntp,
             acc, src_v, dst_v, rows_a, rows_b, cnt_loc, sem_a, sem_b):
    cid = lax.axis_index("c")
    sid = lax.axis_index("s")
    wid = cid * NS + sid

    zero16 = jnp.zeros((16,), jnp.float32)
    one16 = jnp.ones((16,), jnp.float32)

    def zero_cnt():
        def _zcnt(i, carry):
            cnt_loc[pl.ds(i * 16, 16)] = zero16
            return carry
        lax.fori_loop(0, ND // 16, _zcnt, 0)

    def zero_acc():
        def _z(k, carry):
            pltpu.sync_copy(zrows, acc.at[pl.ds(sid * NPT + k * 125, 125)])
            return carry
        lax.fori_loop(0, NPT // 125, _z, 0)
        plsc.subcore_barrier()

    def do_rel(msg, s2d, d2d):
        row0 = cid * (EROWS // 2) + sid * RPT
        pltpu.sync_copy(s2d.at[pl.ds(row0, RPT)], src_v)
        pltpu.sync_copy(d2d.at[pl.ds(row0, RPT)], dst_v)

        def _cnt(ci):
            def _cl(j, c2):
                idx = dst_v[ci, pl.ds(j * 16, 16)]
                plsc.addupdate_scatter(cnt_loc, [idx], one16)
                return c2
            lax.fori_loop(0, 8, _cl, 0)

        pltpu.async_copy(msg.at[src_v.at[0]], rows_a, sem_a)

        def _outer(o, carry):
            ca = 2 * o
            pltpu.make_async_copy(msg.at[src_v.at[ca]], rows_a, sem_a).wait()
            pltpu.async_copy(msg.at[src_v.at[ca + 1]], rows_b, sem_b)
            pltpu.sync_copy(rows_a, acc.at[dst_v.at[ca]], add=True)
            _cnt(ca)
            pltpu.make_async_copy(
                msg.at[src_v.at[ca + 1]], rows_b, sem_b).wait()

            @pl.when(o < RPT // 2 - 1)
            def _():
                pltpu.async_copy(msg.at[src_v.at[ca + 2]], rows_a, sem_a)
            pltpu.sync_copy(rows_b, acc.at[dst_v.at[ca + 1]], add=True)
            _cnt(ca + 1)
            return carry
        lax.fori_loop(0, RPT // 2, _outer, 0)

    def dump(out):
        plsc.subcore_barrier()
        pltpu.sync_copy(acc.at[pl.ds(sid * NPT, NPT)],
                        out.at[cid].at[pl.ds(sid * NPT, NPT)])
        plsc.subcore_barrier()

    zero_cnt()
    zero_acc()
    do_rel(m_dd, s_dd, d_dd)
    do_rel(m_td, s_td, d_td)
    dump(aggd)
    pltpu.sync_copy(cnt_loc, cntd.at[wid])
    zero_cnt()
    zero_acc()
    do_rel(m_dt, s_dt, d_dt)
    do_rel(m_pp, s_pp, d_pp)
    dump(aggp)
    pltpu.sync_copy(cnt_loc, cntp.at[wid])


_sc_agg = functools.partial(
    pl.kernel,
    out_type=[jax.ShapeDtypeStruct((NC, ND, D), jnp.float32),
              jax.ShapeDtypeStruct((NC, ND, D), jnp.float32),
              jax.ShapeDtypeStruct((NC * NS, ND), jnp.float32),
              jax.ShapeDtypeStruct((NC * NS, ND), jnp.float32)],
    mesh=plsc.VectorSubcoreMesh(core_axis_name="c", subcore_axis_name="s",
                                num_cores=NC, num_subcores=NS),
    compiler_params=pltpu.CompilerParams(use_tc_tiling_on_sc=False,
                                         needs_layout_passes=False),
    scratch_types=[
        pltpu.VMEM_SHARED((ND, D), jnp.float32),
        pltpu.VMEM((RPT, 128), jnp.int32),
        pltpu.VMEM((RPT, 128), jnp.int32),
        pltpu.VMEM((128, D), jnp.float32),
        pltpu.VMEM((128, D), jnp.float32),
        pltpu.VMEM((ND,), jnp.float32),
        pltpu.SemaphoreType.DMA,
        pltpu.SemaphoreType.DMA,
    ],
)(_sc_body)


def kernel(x_drug, x_protein, edge_dd, edge_dt, edge_td, edge_pp,
           Wp_drug, bp_drug, Wp_protein, bp_protein,
           W_rel, W_self, ln_gamma, ln_beta):
    h_d = _input_proj(x_drug, Wp_drug.T, bp_drug[None, :])
    h_p = _input_proj(x_protein, Wp_protein.T, bp_protein[None, :])

    def split(e):
        e = e.astype(jnp.int32)
        return e[0].reshape(EROWS, 128), e[1].reshape(EROWS, 128)

    s_dd, d_dd = split(edge_dd)
    s_dt, d_dt = split(edge_dt)
    s_td, d_td = split(edge_td)
    s_pp, d_pp = split(edge_pp)
    zrows = jnp.zeros((125, D), jnp.float32)

    for l in range(2):
        w_d = jnp.concatenate(
            [W_rel[l, 0].T, W_rel[l, 1].T, W_self[l, 0].T], axis=1)
        w_p = jnp.concatenate(
            [W_rel[l, 2].T, W_rel[l, 3].T, W_self[l, 1].T], axis=1)
        m_dd, m_dt, self_d = _msg(h_d, w_d)
        m_td, m_pp, self_p = _msg(h_p, w_p)
        aggd, aggp, cntd, cntp = _sc_agg(
            m_dd, m_dt, m_td, m_pp,
            s_dd, d_dd, s_dt, d_dt, s_td, d_td, s_pp, d_pp, zrows)
        h_d = _combine(h_d, self_d, aggd, cntd.T,
                       ln_gamma[l, 0][None, :], ln_beta[l, 0][None, :])
        h_p = _combine(h_p, self_p, aggp, cntp.T,
                       ln_gamma[l, 1][None, :], ln_beta[l, 1][None, :])

    return jnp.concatenate([h_d, h_p], axis=0)


# speedup vs baseline: 8.1818x; 1.0748x over previous
import functools

import jax
import jax.numpy as jnp
from jax import lax
from jax.experimental import pallas as pl
from jax.experimental.pallas import tpu as pltpu
from jax.experimental.pallas import tpu_sc as plsc

ND = 10000
D = 128
DIN = 256
E = 131072
NC = 2
NS = 16
CH = 64
NBUF = 4
EROWS = E // CH
RPT = EROWS // (NC * NS)
NPT = ND // NS


def _proj_body(x_ref, w_ref, b_ref, o_ref):
    o_ref[...] = jnp.maximum(
        jnp.dot(x_ref[...], w_ref[...], preferred_element_type=jnp.float32)
        + b_ref[...], 0.0)


def _input_proj(x, w_t, b):
    blk = 2000
    return pl.pallas_call(
        _proj_body,
        grid=(ND // blk,),
        in_specs=[pl.BlockSpec((blk, DIN), lambda i: (i, 0)),
                  pl.BlockSpec((DIN, D), lambda i: (0, 0)),
                  pl.BlockSpec((1, D), lambda i: (0, 0))],
        out_specs=pl.BlockSpec((blk, D), lambda i: (i, 0)),
        out_shape=jax.ShapeDtypeStruct((ND, D), jnp.float32),
    )(x, w_t, b)


def _msg_body(h_ref, w_ref, oa_ref, ob_ref, os_ref):
    r = jnp.dot(h_ref[...], w_ref[...], preferred_element_type=jnp.float32)
    oa_ref[...] = r[:, :D]
    ob_ref[...] = r[:, D:2 * D]
    os_ref[...] = r[:, 2 * D:]


def _msg(h, w_cat):
    blk = 2000
    return pl.pallas_call(
        _msg_body,
        grid=(ND // blk,),
        in_specs=[pl.BlockSpec((blk, D), lambda i: (i, 0)),
                  pl.BlockSpec((D, 3 * D), lambda i: (0, 0))],
        out_specs=[pl.BlockSpec((blk, D), lambda i: (i, 0))] * 3,
        out_shape=[jax.ShapeDtypeStruct((ND, D), jnp.float32)] * 3,
    )(h, w_cat)


def _combine_body(h_ref, s_ref, agg_ref, cnt_ref, g_ref, b_ref, o_ref):
    agg = agg_ref[0] + agg_ref[1]
    cnt = jnp.sum(cnt_ref[...], axis=1, keepdims=True)
    t = jnp.maximum(s_ref[...] + agg / jnp.maximum(cnt, 1.0), 0.0) + h_ref[...]
    mu = jnp.mean(t, axis=-1, keepdims=True)
    var = jnp.mean((t - mu) ** 2, axis=-1, keepdims=True)
    o_ref[...] = (t - mu) * lax.rsqrt(var + 1e-5) * g_ref[...] + b_ref[...]


def _combine(h, s, agg, cnt, g, b):
    blk = 2000
    return pl.pallas_call(
        _combine_body,
        grid=(ND // blk,),
        in_specs=[pl.BlockSpec((blk, D), lambda i: (i, 0)),
                  pl.BlockSpec((blk, D), lambda i: (i, 0)),
                  pl.BlockSpec((2, blk, D), lambda i: (0, i, 0)),
                  pl.BlockSpec((blk, NC * NS), lambda i: (i, 0)),
                  pl.BlockSpec((1, D), lambda i: (0, 0)),
                  pl.BlockSpec((1, D), lambda i: (0, 0))],
        out_specs=pl.BlockSpec((blk, D), lambda i: (i, 0)),
        out_shape=jax.ShapeDtypeStruct((ND, D), jnp.float32),
    )(h, s, agg, cnt, g, b)


def _sc_body(m_dd, m_dt, m_td, m_pp,
             s_dd, d_dd, s_dt, d_dt, s_td, d_td, s_pp, d_pp, zrows,
             aggd, aggp, cntd, cntp,
             acc, src_v, dst_v, rows0, rows1, rows2, rows3, cnt_loc,
             g0, g1, g2, g3, ss0, ss1, ss2, ss3):
    rows = (rows0, rows1, rows2, rows3)
    gsem = (g0, g1, g2, g3)
    ssem = (ss0, ss1, ss2, ss3)
    cid = lax.axis_index("c")
    sid = lax.axis_index("s")
    wid = cid * NS + sid

    zero16 = jnp.zeros((16,), jnp.float32)
    one16 = jnp.ones((16,), jnp.float32)

    def zero_cnt():
        def _zcnt(i, carry):
            cnt_loc[pl.ds(i * 16, 16)] = zero16
            return carry
        lax.fori_loop(0, ND // 16, _zcnt, 0)

    def zero_acc():
        def _z(k, carry):
            pltpu.sync_copy(zrows, acc.at[pl.ds(sid * NPT + k * 125, 125)])
            return carry
        lax.fori_loop(0, NPT // 125, _z, 0)
        plsc.subcore_barrier()

    def do_rel(msg, s2d, d2d):
        row0 = cid * (EROWS // 2) + sid * RPT
        pltpu.sync_copy(s2d.at[pl.ds(row0, RPT)], src_v)
        pltpu.sync_copy(d2d.at[pl.ds(row0, RPT)], dst_v)

        for b in range(NBUF):
            pltpu.async_copy(msg.at[src_v.at[b]], rows[b], gsem[b])

        def _grp(o, carry):
            c0 = NBUF * o
            sdesc = []
            for b in range(NBUF):
                c = c0 + b
                pltpu.make_async_copy(
                    msg.at[src_v.at[c]], rows[b], gsem[b]).wait()
                sdesc.append(pltpu.async_copy(
                    rows[b], acc.at[dst_v.at[c]], ssem[b], add=True))
                for j in range(CH // 16):
                    idx = dst_v[c, pl.ds(j * 16, 16)]
                    plsc.addupdate_scatter(cnt_loc, [idx], one16)
            for b in range(NBUF):
                sdesc[b].wait()

                @pl.when(o < RPT // NBUF - 1)
                def _():
                    pltpu.async_copy(
                        msg.at[src_v.at[c0 + b + NBUF]], rows[b], gsem[b])
            return carry
        lax.fori_loop(0, RPT // NBUF, _grp, 0)

    def dump(out):
        plsc.subcore_barrier()
        pltpu.sync_copy(acc.at[pl.ds(sid * NPT, NPT)],
                        out.at[cid].at[pl.ds(sid * NPT, NPT)])
        plsc.subcore_barrier()

    zero_cnt()
    zero_acc()
    do_rel(m_dd, s_dd, d_dd)
    do_rel(m_td, s_td, d_td)
    dump(aggd)
    pltpu.sync_copy(cnt_loc, cntd.at[wid])
    zero_cnt()
    zero_acc()
    do_rel(m_dt, s_dt, d_dt)
    do_rel(m_pp, s_pp, d_pp)
    dump(aggp)
    pltpu.sync_copy(cnt_loc, cntp.at[wid])


_sc_agg = functools.partial(
    pl.kernel,
    out_type=[jax.ShapeDtypeStruct((NC, ND, D), jnp.float32),
              jax.ShapeDtypeStruct((NC, ND, D), jnp.float32),
              jax.ShapeDtypeStruct((NC * NS, ND), jnp.float32),
              jax.ShapeDtypeStruct((NC * NS, ND), jnp.float32)],
    mesh=plsc.VectorSubcoreMesh(core_axis_name="c", subcore_axis_name="s",
                                num_cores=NC, num_subcores=NS),
    compiler_params=pltpu.CompilerParams(use_tc_tiling_on_sc=False,
                                         needs_layout_passes=False),
    scratch_types=[
        pltpu.VMEM_SHARED((ND, D), jnp.float32),
        pltpu.VMEM((RPT, CH), jnp.int32),
        pltpu.VMEM((RPT, CH), jnp.int32),
        pltpu.VMEM((CH, D), jnp.float32),
        pltpu.VMEM((CH, D), jnp.float32),
        pltpu.VMEM((CH, D), jnp.float32),
        pltpu.VMEM((CH, D), jnp.float32),
        pltpu.VMEM((ND,), jnp.float32),
        pltpu.SemaphoreType.DMA,
        pltpu.SemaphoreType.DMA,
        pltpu.SemaphoreType.DMA,
        pltpu.SemaphoreType.DMA,
        pltpu.SemaphoreType.DMA,
        pltpu.SemaphoreType.DMA,
        pltpu.SemaphoreType.DMA,
        pltpu.SemaphoreType.DMA,
    ],
)(_sc_body)


def kernel(x_drug, x_protein, edge_dd, edge_dt, edge_td, edge_pp,
           Wp_drug, bp_drug, Wp_protein, bp_protein,
           W_rel, W_self, ln_gamma, ln_beta):
    h_d = _input_proj(x_drug, Wp_drug.T, bp_drug[None, :])
    h_p = _input_proj(x_protein, Wp_protein.T, bp_protein[None, :])

    def split(e):
        e = e.astype(jnp.int32)
        return e[0].reshape(EROWS, CH), e[1].reshape(EROWS, CH)

    s_dd, d_dd = split(edge_dd)
    s_dt, d_dt = split(edge_dt)
    s_td, d_td = split(edge_td)
    s_pp, d_pp = split(edge_pp)
    zrows = jnp.zeros((125, D), jnp.float32)

    for l in range(2):
        w_d = jnp.concatenate(
            [W_rel[l, 0].T, W_rel[l, 1].T, W_self[l, 0].T], axis=1)
        w_p = jnp.concatenate(
            [W_rel[l, 2].T, W_rel[l, 3].T, W_self[l, 1].T], axis=1)
        m_dd, m_dt, self_d = _msg(h_d, w_d)
        m_td, m_pp, self_p = _msg(h_p, w_p)
        aggd, aggp, cntd, cntp = _sc_agg(
            m_dd, m_dt, m_td, m_pp,
            s_dd, d_dd, s_dt, d_dt, s_td, d_td, s_pp, d_pp, zrows)
        h_d = _combine(h_d, self_d, aggd, cntd.T,
                       ln_gamma[l, 0][None, :], ln_beta[l, 0][None, :])
        h_p = _combine(h_p, self_p, aggp, cntp.T,
                       ln_gamma[l, 1][None, :], ln_beta[l, 1][None, :])

    return jnp.concatenate([h_d, h_p], axis=0)


# speedup vs baseline: 8.7334x; 1.0674x over previous
import functools

import jax
import jax.numpy as jnp
from jax import lax
from jax.experimental import pallas as pl
from jax.experimental.pallas import tpu as pltpu
from jax.experimental.pallas import tpu_sc as plsc

ND = 10000
D = 128
DIN = 256
E = 131072
NC = 2
NS = 16
CH = 64
NBUF = 4
EROWS = E // CH
RPT = EROWS // (NC * NS)
NPT = ND // NS


def _split3(r, oa_ref, ob_ref, os_ref):
    oa_ref[...] = r[:, :D]
    ob_ref[...] = r[:, D:2 * D]
    os_ref[...] = r[:, 2 * D:]


def _projmsg_body(x_ref, wp_ref, bp_ref, w_ref, h_ref, oa_ref, ob_ref,
                  os_ref):
    h = jnp.maximum(
        jnp.dot(x_ref[...], wp_ref[...], preferred_element_type=jnp.float32)
        + bp_ref[...], 0.0)
    h_ref[...] = h
    _split3(jnp.dot(h, w_ref[...], preferred_element_type=jnp.float32),
            oa_ref, ob_ref, os_ref)


def _projmsg(x, wp_t, bp, w_cat):
    blk = 2000
    return pl.pallas_call(
        _projmsg_body,
        grid=(ND // blk,),
        in_specs=[pl.BlockSpec((blk, DIN), lambda i: (i, 0)),
                  pl.BlockSpec((DIN, D), lambda i: (0, 0)),
                  pl.BlockSpec((1, D), lambda i: (0, 0)),
                  pl.BlockSpec((D, 3 * D), lambda i: (0, 0))],
        out_specs=[pl.BlockSpec((blk, D), lambda i: (i, 0))] * 4,
        out_shape=[jax.ShapeDtypeStruct((ND, D), jnp.float32)] * 4,
    )(x, wp_t, bp, w_cat)


def _combine_val(h_ref, s_ref, agg_ref, cnt_ref, g_ref, b_ref):
    agg = agg_ref[0] + agg_ref[1]
    cnt = jnp.sum(cnt_ref[...], axis=1, keepdims=True)
    t = jnp.maximum(s_ref[...] + agg / jnp.maximum(cnt, 1.0), 0.0) + h_ref[...]
    mu = jnp.mean(t, axis=-1, keepdims=True)
    var = jnp.mean((t - mu) ** 2, axis=-1, keepdims=True)
    return (t - mu) * lax.rsqrt(var + 1e-5) * g_ref[...] + b_ref[...]


def _combine_body(h_ref, s_ref, agg_ref, cnt_ref, g_ref, b_ref, o_ref):
    o_ref[...] = _combine_val(h_ref, s_ref, agg_ref, cnt_ref, g_ref, b_ref)


def _combmsg_body(h_ref, s_ref, agg_ref, cnt_ref, g_ref, b_ref, w_ref,
                  hn_ref, oa_ref, ob_ref, os_ref):
    y = _combine_val(h_ref, s_ref, agg_ref, cnt_ref, g_ref, b_ref)
    hn_ref[...] = y
    _split3(jnp.dot(y, w_ref[...], preferred_element_type=jnp.float32),
            oa_ref, ob_ref, os_ref)


def _combmsg(h, sf, agg, cnt, g, b, w_cat):
    blk = 2000
    return pl.pallas_call(
        _combmsg_body,
        grid=(ND // blk,),
        in_specs=[pl.BlockSpec((blk, D), lambda i: (i, 0)),
                  pl.BlockSpec((blk, D), lambda i: (i, 0)),
                  pl.BlockSpec((2, blk, D), lambda i: (0, i, 0)),
                  pl.BlockSpec((blk, NC * NS), lambda i: (i, 0)),
                  pl.BlockSpec((1, D), lambda i: (0, 0)),
                  pl.BlockSpec((1, D), lambda i: (0, 0)),
                  pl.BlockSpec((D, 3 * D), lambda i: (0, 0))],
        out_specs=[pl.BlockSpec((blk, D), lambda i: (i, 0))] * 4,
        out_shape=[jax.ShapeDtypeStruct((ND, D), jnp.float32)] * 4,
    )(h, sf, agg, cnt, g, b, w_cat)


def _combine(h, s, agg, cnt, g, b):
    blk = 2000
    return pl.pallas_call(
        _combine_body,
        grid=(ND // blk,),
        in_specs=[pl.BlockSpec((blk, D), lambda i: (i, 0)),
                  pl.BlockSpec((blk, D), lambda i: (i, 0)),
                  pl.BlockSpec((2, blk, D), lambda i: (0, i, 0)),
                  pl.BlockSpec((blk, NC * NS), lambda i: (i, 0)),
                  pl.BlockSpec((1, D), lambda i: (0, 0)),
                  pl.BlockSpec((1, D), lambda i: (0, 0))],
        out_specs=pl.BlockSpec((blk, D), lambda i: (i, 0)),
        out_shape=jax.ShapeDtypeStruct((ND, D), jnp.float32),
    )(h, s, agg, cnt, g, b)


def _sc_body(m_dd, m_dt, m_td, m_pp,
             s_dd, d_dd, s_dt, d_dt, s_td, d_td, s_pp, d_pp, zrows,
             aggd, aggp, cntd, cntp,
             acc, src_v, dst_v, rows0, rows1, rows2, rows3, cnt_loc,
             g0, g1, g2, g3, ss0, ss1, ss2, ss3):
    rows = (rows0, rows1, rows2, rows3)
    gsem = (g0, g1, g2, g3)
    ssem = (ss0, ss1, ss2, ss3)
    cid = lax.axis_index("c")
    sid = lax.axis_index("s")
    wid = cid * NS + sid

    zero16 = jnp.zeros((16,), jnp.float32)
    one16 = jnp.ones((16,), jnp.float32)

    def zero_cnt():
        def _zcnt(i, carry):
            cnt_loc[pl.ds(i * 16, 16)] = zero16
            return carry
        lax.fori_loop(0, ND // 16, _zcnt, 0)

    def zero_acc():
        def _z(k, carry):
            pltpu.sync_copy(zrows, acc.at[pl.ds(sid * NPT + k * 125, 125)])
            return carry
        lax.fori_loop(0, NPT // 125, _z, 0)
        plsc.subcore_barrier()

    def do_rel(msg, s2d, d2d):
        row0 = cid * (EROWS // 2) + sid * RPT
        pltpu.sync_copy(s2d.at[pl.ds(row0, RPT)], src_v)
        pltpu.sync_copy(d2d.at[pl.ds(row0, RPT)], dst_v)

        for b in range(NBUF):
            pltpu.async_copy(msg.at[src_v.at[b]], rows[b], gsem[b])

        def _grp(o, carry):
            c0 = NBUF * o
            sdesc = []
            for b in range(NBUF):
                c = c0 + b
                pltpu.make_async_copy(
                    msg.at[src_v.at[c]], rows[b], gsem[b]).wait()
                sdesc.append(pltpu.async_copy(
                    rows[b], acc.at[dst_v.at[c]], ssem[b], add=True))
                for j in range(CH // 16):
                    idx = dst_v[c, pl.ds(j * 16, 16)]
                    plsc.addupdate_scatter(cnt_loc, [idx], one16)
            for b in range(NBUF):
                sdesc[b].wait()

                @pl.when(o < RPT // NBUF - 1)
                def _():
                    pltpu.async_copy(
                        msg.at[src_v.at[c0 + b + NBUF]], rows[b], gsem[b])
            return carry
        lax.fori_loop(0, RPT // NBUF, _grp, 0)

    def dump(out):
        plsc.subcore_barrier()
        pltpu.sync_copy(acc.at[pl.ds(sid * NPT, NPT)],
                        out.at[cid].at[pl.ds(sid * NPT, NPT)])
        plsc.subcore_barrier()

    zero_cnt()
    zero_acc()
    do_rel(m_dd, s_dd, d_dd)
    do_rel(m_td, s_td, d_td)
    dump(aggd)
    pltpu.sync_copy(cnt_loc, cntd.at[wid])
    zero_cnt()
    zero_acc()
    do_rel(m_dt, s_dt, d_dt)
    do_rel(m_pp, s_pp, d_pp)
    dump(aggp)
    pltpu.sync_copy(cnt_loc, cntp.at[wid])


_sc_agg = functools.partial(
    pl.kernel,
    out_type=[jax.ShapeDtypeStruct((NC, ND, D), jnp.float32),
              jax.ShapeDtypeStruct((NC, ND, D), jnp.float32),
              jax.ShapeDtypeStruct((NC * NS, ND), jnp.float32),
              jax.ShapeDtypeStruct((NC * NS, ND), jnp.float32)],
    mesh=plsc.VectorSubcoreMesh(core_axis_name="c", subcore_axis_name="s",
                                num_cores=NC, num_subcores=NS),
    compiler_params=pltpu.CompilerParams(use_tc_tiling_on_sc=False,
                                         needs_layout_passes=False),
    scratch_types=[
        pltpu.VMEM_SHARED((ND, D), jnp.float32),
        pltpu.VMEM((RPT, CH), jnp.int32),
        pltpu.VMEM((RPT, CH), jnp.int32),
        pltpu.VMEM((CH, D), jnp.float32),
        pltpu.VMEM((CH, D), jnp.float32),
        pltpu.VMEM((CH, D), jnp.float32),
        pltpu.VMEM((CH, D), jnp.float32),
        pltpu.VMEM((ND,), jnp.float32),
        pltpu.SemaphoreType.DMA,
        pltpu.SemaphoreType.DMA,
        pltpu.SemaphoreType.DMA,
        pltpu.SemaphoreType.DMA,
        pltpu.SemaphoreType.DMA,
        pltpu.SemaphoreType.DMA,
        pltpu.SemaphoreType.DMA,
        pltpu.SemaphoreType.DMA,
    ],
)(_sc_body)


def kernel(x_drug, x_protein, edge_dd, edge_dt, edge_td, edge_pp,
           Wp_drug, bp_drug, Wp_protein, bp_protein,
           W_rel, W_self, ln_gamma, ln_beta):
    def split(e):
        e = e.astype(jnp.int32)
        return e[0].reshape(EROWS, CH), e[1].reshape(EROWS, CH)

    s_dd, d_dd = split(edge_dd)
    s_dt, d_dt = split(edge_dt)
    s_td, d_td = split(edge_td)
    s_pp, d_pp = split(edge_pp)
    zrows = jnp.zeros((125, D), jnp.float32)

    w_d0 = jnp.concatenate(
        [W_rel[0, 0].T, W_rel[0, 1].T, W_self[0, 0].T], axis=1)
    w_p0 = jnp.concatenate(
        [W_rel[0, 2].T, W_rel[0, 3].T, W_self[0, 1].T], axis=1)
    w_d1 = jnp.concatenate(
        [W_rel[1, 0].T, W_rel[1, 1].T, W_self[1, 0].T], axis=1)
    w_p1 = jnp.concatenate(
        [W_rel[1, 2].T, W_rel[1, 3].T, W_self[1, 1].T], axis=1)

    h_d, m_dd, m_dt, self_d = _projmsg(x_drug, Wp_drug.T, bp_drug[None, :],
                                       w_d0)
    h_p, m_td, m_pp, self_p = _projmsg(x_protein, Wp_protein.T,
                                       bp_protein[None, :], w_p0)
    aggd, aggp, cntd, cntp = _sc_agg(
        m_dd, m_dt, m_td, m_pp,
        s_dd, d_dd, s_dt, d_dt, s_td, d_td, s_pp, d_pp, zrows)
    h_d, m_dd, m_dt, self_d = _combmsg(
        h_d, self_d, aggd, cntd.T,
        ln_gamma[0, 0][None, :], ln_beta[0, 0][None, :], w_d1)
    h_p, m_td, m_pp, self_p = _combmsg(
        h_p, self_p, aggp, cntp.T,
        ln_gamma[0, 1][None, :], ln_beta[0, 1][None, :], w_p1)
    aggd, aggp, cntd, cntp = _sc_agg(
        m_dd, m_dt, m_td, m_pp,
        s_dd, d_dd, s_dt, d_dt, s_td, d_td, s_pp, d_pp, zrows)
    h_d = _combine(h_d, self_d, aggd, cntd.T,
                   ln_gamma[1, 0][None, :], ln_beta[1, 0][None, :])
    h_p = _combine(h_p, self_p, aggp, cntp.T,
                   ln_gamma[1, 1][None, :], ln_beta[1, 1][None, :])

    return jnp.concatenate([h_d, h_p], axis=0)


# speedup vs baseline: 9.0986x; 1.0418x over previous
import functools

import jax
import jax.numpy as jnp
from jax import lax
from jax.experimental import pallas as pl
from jax.experimental.pallas import tpu as pltpu
from jax.experimental.pallas import tpu_sc as plsc

ND = 10000
D = 128
DIN = 256
E = 131072
NC = 2
NS = 16
CH = 64
NBUF = 4
EROWS = E // CH
RPT = EROWS // (NC * NS)
NPT = ND // NS


def _split3(r, oa_ref, ob_ref, os_ref):
    oa_ref[...] = r[:, :D]
    ob_ref[...] = r[:, D:2 * D]
    os_ref[...] = r[:, 2 * D:]


def _projmsg_body(x_ref, wp_ref, bp_ref, w_ref, h_ref, oa_ref, ob_ref,
                  os_ref):
    h = jnp.maximum(
        jnp.dot(x_ref[...], wp_ref[...], preferred_element_type=jnp.float32)
        + bp_ref[...], 0.0)
    h_ref[...] = h
    _split3(jnp.dot(h, w_ref[...], preferred_element_type=jnp.float32),
            oa_ref, ob_ref, os_ref)


def _projmsg(x, wp_t, bp, w_cat):
    blk = 2000
    return pl.pallas_call(
        _projmsg_body,
        grid=(ND // blk,),
        in_specs=[pl.BlockSpec((blk, DIN), lambda i: (i, 0)),
                  pl.BlockSpec((DIN, D), lambda i: (0, 0)),
                  pl.BlockSpec((1, D), lambda i: (0, 0)),
                  pl.BlockSpec((D, 3 * D), lambda i: (0, 0))],
        out_specs=[pl.BlockSpec((blk, D), lambda i: (i, 0))] * 4,
        out_shape=[jax.ShapeDtypeStruct((ND, D), jnp.float32)] * 4,
    )(x, wp_t, bp, w_cat)


def _combine_val(h_ref, s_ref, agg_ref, cnt_ref, g_ref, b_ref):
    agg = agg_ref[0] + agg_ref[1]
    cnt = jnp.sum(cnt_ref[...], axis=1, keepdims=True)
    t = jnp.maximum(s_ref[...] + agg / jnp.maximum(cnt, 1.0), 0.0) + h_ref[...]
    mu = jnp.mean(t, axis=-1, keepdims=True)
    var = jnp.mean((t - mu) ** 2, axis=-1, keepdims=True)
    return (t - mu) * lax.rsqrt(var + 1e-5) * g_ref[...] + b_ref[...]


def _combine_body(h_ref, s_ref, agg_ref, cnt_ref, g_ref, b_ref, o_ref):
    o_ref[...] = _combine_val(h_ref, s_ref, agg_ref, cnt_ref, g_ref, b_ref)


def _combmsg_body(h_ref, s_ref, agg_ref, cnt_ref, g_ref, b_ref, w_ref,
                  hn_ref, oa_ref, ob_ref, os_ref):
    y = _combine_val(h_ref, s_ref, agg_ref, cnt_ref, g_ref, b_ref)
    hn_ref[...] = y
    _split3(jnp.dot(y, w_ref[...], preferred_element_type=jnp.float32),
            oa_ref, ob_ref, os_ref)


def _combmsg(h, sf, agg, cnt, g, b, w_cat):
    blk = 2000
    return pl.pallas_call(
        _combmsg_body,
        grid=(ND // blk,),
        in_specs=[pl.BlockSpec((blk, D), lambda i: (i, 0)),
                  pl.BlockSpec((blk, D), lambda i: (i, 0)),
                  pl.BlockSpec((2, blk, D), lambda i: (0, i, 0)),
                  pl.BlockSpec((blk, NC * NS), lambda i: (i, 0)),
                  pl.BlockSpec((1, D), lambda i: (0, 0)),
                  pl.BlockSpec((1, D), lambda i: (0, 0)),
                  pl.BlockSpec((D, 3 * D), lambda i: (0, 0))],
        out_specs=[pl.BlockSpec((blk, D), lambda i: (i, 0))] * 4,
        out_shape=[jax.ShapeDtypeStruct((ND, D), jnp.float32)] * 4,
    )(h, sf, agg, cnt, g, b, w_cat)


def _combine(h, s, agg, cnt, g, b):
    blk = 2000
    return pl.pallas_call(
        _combine_body,
        grid=(ND // blk,),
        in_specs=[pl.BlockSpec((blk, D), lambda i: (i, 0)),
                  pl.BlockSpec((blk, D), lambda i: (i, 0)),
                  pl.BlockSpec((2, blk, D), lambda i: (0, i, 0)),
                  pl.BlockSpec((blk, NC * NS), lambda i: (i, 0)),
                  pl.BlockSpec((1, D), lambda i: (0, 0)),
                  pl.BlockSpec((1, D), lambda i: (0, 0))],
        out_specs=pl.BlockSpec((blk, D), lambda i: (i, 0)),
        out_shape=jax.ShapeDtypeStruct((ND, D), jnp.float32),
    )(h, s, agg, cnt, g, b)


def _make_sc_phase(with_cnt):
    def body(*refs):
        if with_cnt:
            (m_a, m_b, s_a, d_a, s_b, d_b, zrows, agg, cnt,
             acc, src_v, dst_v, rows0, rows1, rows2, rows3, cnt_loc,
             g0, g1, g2, g3, ss0, ss1, ss2, ss3) = refs
        else:
            (m_a, m_b, s_a, d_a, s_b, d_b, zrows, agg,
             acc, src_v, dst_v, rows0, rows1, rows2, rows3,
             g0, g1, g2, g3, ss0, ss1, ss2, ss3) = refs
        rows = (rows0, rows1, rows2, rows3)
        gsem = (g0, g1, g2, g3)
        ssem = (ss0, ss1, ss2, ss3)
        cid = lax.axis_index("c")
        sid = lax.axis_index("s")

        zero16 = jnp.zeros((16,), jnp.float32)
        one16 = jnp.ones((16,), jnp.float32)

        if with_cnt:
            def _zcnt(i, carry):
                cnt_loc[pl.ds(i * 16, 16)] = zero16
                return carry
            lax.fori_loop(0, ND // 16, _zcnt, 0)

        def _z(k, carry):
            pltpu.sync_copy(zrows, acc.at[pl.ds(sid * NPT + k * 125, 125)])
            return carry
        lax.fori_loop(0, NPT // 125, _z, 0)
        plsc.subcore_barrier()

        def do_rel(msg, s2d, d2d):
            row0 = cid * (EROWS // 2) + sid * RPT
            pltpu.sync_copy(s2d.at[pl.ds(row0, RPT)], src_v)
            pltpu.sync_copy(d2d.at[pl.ds(row0, RPT)], dst_v)

            for b in range(NBUF):
                pltpu.async_copy(msg.at[src_v.at[b]], rows[b], gsem[b])

            def _grp(o, carry):
                c0 = NBUF * o
                sdesc = []
                for b in range(NBUF):
                    c = c0 + b
                    pltpu.make_async_copy(
                        msg.at[src_v.at[c]], rows[b], gsem[b]).wait()
                    sdesc.append(pltpu.async_copy(
                        rows[b], acc.at[dst_v.at[c]], ssem[b], add=True))
                    if with_cnt:
                        for j in range(CH // 16):
                            idx = dst_v[c, pl.ds(j * 16, 16)]
                            plsc.addupdate_scatter(cnt_loc, [idx], one16)
                for b in range(NBUF):
                    sdesc[b].wait()

                    @pl.when(o < RPT // NBUF - 1)
                    def _():
                        pltpu.async_copy(
                            msg.at[src_v.at[c0 + b + NBUF]], rows[b], gsem[b])
                return carry
            lax.fori_loop(0, RPT // NBUF, _grp, 0)

        do_rel(m_a, s_a, d_a)
        do_rel(m_b, s_b, d_b)

        plsc.subcore_barrier()
        pltpu.sync_copy(acc.at[pl.ds(sid * NPT, NPT)],
                        agg.at[cid].at[pl.ds(sid * NPT, NPT)])
        if with_cnt:
            pltpu.sync_copy(cnt_loc, cnt.at[cid * NS + sid])

    out_type = [jax.ShapeDtypeStruct((NC, ND, D), jnp.float32)]
    scratch = [
        pltpu.VMEM_SHARED((ND, D), jnp.float32),
        pltpu.VMEM((RPT, CH), jnp.int32),
        pltpu.VMEM((RPT, CH), jnp.int32),
        pltpu.VMEM((CH, D), jnp.float32),
        pltpu.VMEM((CH, D), jnp.float32),
        pltpu.VMEM((CH, D), jnp.float32),
        pltpu.VMEM((CH, D), jnp.float32),
    ]
    if with_cnt:
        out_type.append(jax.ShapeDtypeStruct((NC * NS, ND), jnp.float32))
        scratch.append(pltpu.VMEM((ND,), jnp.float32))
    scratch += [pltpu.SemaphoreType.DMA] * 8
    return pl.kernel(
        body,
        out_type=out_type,
        mesh=plsc.VectorSubcoreMesh(core_axis_name="c", subcore_axis_name="s",
                                    num_cores=NC, num_subcores=NS),
        compiler_params=pltpu.CompilerParams(use_tc_tiling_on_sc=False,
                                             needs_layout_passes=False),
        scratch_types=scratch,
    )


_sc_phase_cnt = _make_sc_phase(True)
_sc_phase_nocnt = _make_sc_phase(False)


def kernel(x_drug, x_protein, edge_dd, edge_dt, edge_td, edge_pp,
           Wp_drug, bp_drug, Wp_protein, bp_protein,
           W_rel, W_self, ln_gamma, ln_beta):
    def split(e):
        e = e.astype(jnp.int32)
        return e[0].reshape(EROWS, CH), e[1].reshape(EROWS, CH)

    s_dd, d_dd = split(edge_dd)
    s_dt, d_dt = split(edge_dt)
    s_td, d_td = split(edge_td)
    s_pp, d_pp = split(edge_pp)
    zrows = jnp.zeros((125, D), jnp.float32)

    w_d0 = jnp.concatenate(
        [W_rel[0, 0].T, W_rel[0, 1].T, W_self[0, 0].T], axis=1)
    w_p0 = jnp.concatenate(
        [W_rel[0, 2].T, W_rel[0, 3].T, W_self[0, 1].T], axis=1)
    w_d1 = jnp.concatenate(
        [W_rel[1, 0].T, W_rel[1, 1].T, W_self[1, 0].T], axis=1)
    w_p1 = jnp.concatenate(
        [W_rel[1, 2].T, W_rel[1, 3].T, W_self[1, 1].T], axis=1)

    h_d, m_dd, m_dt, self_d = _projmsg(x_drug, Wp_drug.T, bp_drug[None, :],
                                       w_d0)
    h_p, m_td, m_pp, self_p = _projmsg(x_protein, Wp_protein.T,
                                       bp_protein[None, :], w_p0)
    aggd, cntd = _sc_phase_cnt(m_dd, m_td, s_dd, d_dd, s_td, d_td, zrows)
    aggp, cntp = _sc_phase_cnt(m_dt, m_pp, s_dt, d_dt, s_pp, d_pp, zrows)
    cntd_t, cntp_t = cntd.T, cntp.T
    h_d, m_dd, m_dt, self_d = _combmsg(
        h_d, self_d, aggd, cntd_t,
        ln_gamma[0, 0][None, :], ln_beta[0, 0][None, :], w_d1)
    h_p, m_td, m_pp, self_p = _combmsg(
        h_p, self_p, aggp, cntp_t,
        ln_gamma[0, 1][None, :], ln_beta[0, 1][None, :], w_p1)
    aggd = _sc_phase_nocnt(m_dd, m_td, s_dd, d_dd, s_td, d_td, zrows)[0]
    aggp = _sc_phase_nocnt(m_dt, m_pp, s_dt, d_dt, s_pp, d_pp, zrows)[0]
    h_d = _combine(h_d, self_d, aggd, cntd_t,
                   ln_gamma[1, 0][None, :], ln_beta[1, 0][None, :])
    h_p = _combine(h_p, self_p, aggp, cntp_t,
                   ln_gamma[1, 1][None, :], ln_beta[1, 1][None, :])

    return jnp.concatenate([h_d, h_p], axis=0)


# speedup vs baseline: 9.1045x; 1.0006x over previous
import functools

import jax
import jax.numpy as jnp
from jax import lax
from jax.experimental import pallas as pl
from jax.experimental.pallas import tpu as pltpu
from jax.experimental.pallas import tpu_sc as plsc

ND = 10000
D = 128
DIN = 256
E = 131072
NC = 2
NS = 16
CH = 64
NBUF = 4
EROWS = E // CH
RPT = EROWS // (NC * NS)
NPT = ND // NS


def _split3(r, oa_ref, ob_ref, os_ref):
    oa_ref[...] = r[:, :D]
    ob_ref[...] = r[:, D:2 * D]
    os_ref[...] = r[:, 2 * D:]


def _projmsg_body(x_ref, wp_ref, bp_ref, w_ref, h_ref, oa_ref, ob_ref,
                  os_ref):
    h = jnp.maximum(
        jnp.dot(x_ref[...], wp_ref[...], preferred_element_type=jnp.float32)
        + bp_ref[...], 0.0)
    h_ref[...] = h
    _split3(jnp.dot(h, w_ref[...], preferred_element_type=jnp.float32),
            oa_ref, ob_ref, os_ref)


def _projmsg(x, wp_t, bp, w_cat):
    blk = 2000
    return pl.pallas_call(
        _projmsg_body,
        grid=(ND // blk,),
        in_specs=[pl.BlockSpec((blk, DIN), lambda i: (i, 0)),
                  pl.BlockSpec((DIN, D), lambda i: (0, 0)),
                  pl.BlockSpec((1, D), lambda i: (0, 0)),
                  pl.BlockSpec((D, 3 * D), lambda i: (0, 0))],
        out_specs=[pl.BlockSpec((blk, D), lambda i: (i, 0))] * 4,
        out_shape=[jax.ShapeDtypeStruct((ND, D), jnp.float32)] * 4,
    )(x, wp_t, bp, w_cat)


def _combine_val(h_ref, s_ref, agg_ref, cnt_ref, g_ref, b_ref):
    agg = agg_ref[0] + agg_ref[1]
    cnt = jnp.sum(cnt_ref[...], axis=1, keepdims=True)
    t = jnp.maximum(s_ref[...] + agg / jnp.maximum(cnt, 1.0), 0.0) + h_ref[...]
    mu = jnp.mean(t, axis=-1, keepdims=True)
    var = jnp.mean((t - mu) ** 2, axis=-1, keepdims=True)
    return (t - mu) * lax.rsqrt(var + 1e-5) * g_ref[...] + b_ref[...]


def _combine_body(h_ref, s_ref, agg_ref, cnt_ref, g_ref, b_ref, o_ref):
    o_ref[...] = _combine_val(h_ref, s_ref, agg_ref, cnt_ref, g_ref, b_ref)


def _combmsg_body(h_ref, s_ref, agg_ref, cnt_ref, g_ref, b_ref, w_ref,
                  hn_ref, oa_ref, ob_ref, os_ref):
    y = _combine_val(h_ref, s_ref, agg_ref, cnt_ref, g_ref, b_ref)
    hn_ref[...] = y
    _split3(jnp.dot(y, w_ref[...], preferred_element_type=jnp.float32),
            oa_ref, ob_ref, os_ref)


def _combmsg(h, sf, agg, cnt, g, b, w_cat):
    blk = 2000
    return pl.pallas_call(
        _combmsg_body,
        grid=(ND // blk,),
        in_specs=[pl.BlockSpec((blk, D), lambda i: (i, 0)),
                  pl.BlockSpec((blk, D), lambda i: (i, 0)),
                  pl.BlockSpec((2, blk, D), lambda i: (0, i, 0)),
                  pl.BlockSpec((blk, NC * NS), lambda i: (i, 0)),
                  pl.BlockSpec((1, D), lambda i: (0, 0)),
                  pl.BlockSpec((1, D), lambda i: (0, 0)),
                  pl.BlockSpec((D, 3 * D), lambda i: (0, 0))],
        out_specs=[pl.BlockSpec((blk, D), lambda i: (i, 0))] * 4,
        out_shape=[jax.ShapeDtypeStruct((ND, D), jnp.float32)] * 4,
    )(h, sf, agg, cnt, g, b, w_cat)


def _combine(h, s, agg, cnt, g, b):
    blk = 2000
    return pl.pallas_call(
        _combine_body,
        grid=(ND // blk,),
        in_specs=[pl.BlockSpec((blk, D), lambda i: (i, 0)),
                  pl.BlockSpec((blk, D), lambda i: (i, 0)),
                  pl.BlockSpec((2, blk, D), lambda i: (0, i, 0)),
                  pl.BlockSpec((blk, NC * NS), lambda i: (i, 0)),
                  pl.BlockSpec((1, D), lambda i: (0, 0)),
                  pl.BlockSpec((1, D), lambda i: (0, 0))],
        out_specs=pl.BlockSpec((blk, D), lambda i: (i, 0)),
        out_shape=jax.ShapeDtypeStruct((ND, D), jnp.float32),
    )(h, s, agg, cnt, g, b)


def _make_sc_phase(with_cnt):
    def body(*refs):
        if with_cnt:
            (m_a, m_b, s_a, d_a, s_b, d_b, zrows, agg, cnt,
             acc, src_v, dst_v, rows0, rows1, rows2, rows3, cnt_loc,
             g0, g1, g2, g3, ss0, ss1, ss2, ss3) = refs
        else:
            (m_a, m_b, s_a, d_a, s_b, d_b, zrows, agg,
             acc, src_v, dst_v, rows0, rows1, rows2, rows3,
             g0, g1, g2, g3, ss0, ss1, ss2, ss3) = refs
        rows = (rows0, rows1, rows2, rows3)
        gsem = (g0, g1, g2, g3)
        ssem = (ss0, ss1, ss2, ss3)
        cid = lax.axis_index("c")
        sid = lax.axis_index("s")

        zero16 = jnp.zeros((16,), jnp.float32)
        one16 = jnp.ones((16,), jnp.float32)

        if with_cnt:
            def _zcnt(i, carry):
                cnt_loc[pl.ds(i * 16, 16)] = zero16
                return carry
            lax.fori_loop(0, ND // 16, _zcnt, 0)

        def _z(k, carry):
            pltpu.sync_copy(zrows, acc.at[pl.ds(sid * NPT + k * 125, 125)])
            return carry
        lax.fori_loop(0, NPT // 125, _z, 0)
        plsc.subcore_barrier()

        def do_rel(msg, s2d, d2d):
            row0 = cid * (EROWS // 2) + sid * RPT
            pltpu.sync_copy(s2d.at[pl.ds(row0, RPT)], src_v)
            pltpu.sync_copy(d2d.at[pl.ds(row0, RPT)], dst_v)

            for b in range(NBUF):
                pltpu.async_copy(msg.at[src_v.at[b]], rows[b], gsem[b])

            def _grp(o, carry):
                c0 = NBUF * o
                sdesc = []
                for b in range(NBUF):
                    c = c0 + b
                    pltpu.make_async_copy(
                        msg.at[src_v.at[c]], rows[b], gsem[b]).wait()
                    sdesc.append(pltpu.async_copy(
                        rows[b], acc.at[dst_v.at[c]], ssem[b], add=True))
                    if with_cnt:
                        for j in range(CH // 16):
                            idx = dst_v[c, pl.ds(j * 16, 16)]
                            plsc.addupdate_scatter(cnt_loc, [idx], one16)
                for b in range(NBUF):
                    sdesc[b].wait()

                    @pl.when(o < RPT // NBUF - 1)
                    def _():
                        pltpu.async_copy(
                            msg.at[src_v.at[c0 + b + NBUF]], rows[b], gsem[b])
                return carry
            lax.fori_loop(0, RPT // NBUF, _grp, 0)

        do_rel(m_a, s_a, d_a)
        do_rel(m_b, s_b, d_b)

        plsc.subcore_barrier()
        pltpu.sync_copy(acc.at[pl.ds(sid * NPT, NPT)],
                        agg.at[cid].at[pl.ds(sid * NPT, NPT)])
        if with_cnt:
            pltpu.sync_copy(cnt_loc, cnt.at[cid * NS + sid])

    out_type = [jax.ShapeDtypeStruct((NC, ND, D), jnp.float32)]
    scratch = [
        pltpu.VMEM_SHARED((ND, D), jnp.float32),
        pltpu.VMEM((RPT, CH), jnp.int32),
        pltpu.VMEM((RPT, CH), jnp.int32),
        pltpu.VMEM((CH, D), jnp.float32),
        pltpu.VMEM((CH, D), jnp.float32),
        pltpu.VMEM((CH, D), jnp.float32),
        pltpu.VMEM((CH, D), jnp.float32),
    ]
    if with_cnt:
        out_type.append(jax.ShapeDtypeStruct((NC * NS, ND), jnp.float32))
        scratch.append(pltpu.VMEM((ND,), jnp.float32))
    scratch += [pltpu.SemaphoreType.DMA] * 8
    return pl.kernel(
        body,
        out_type=out_type,
        mesh=plsc.VectorSubcoreMesh(core_axis_name="c", subcore_axis_name="s",
                                    num_cores=NC, num_subcores=NS),
        compiler_params=pltpu.CompilerParams(use_tc_tiling_on_sc=False,
                                             needs_layout_passes=False),
        scratch_types=scratch,
    )


_sc_phase_cnt = _make_sc_phase(True)
_sc_phase_nocnt = _make_sc_phase(False)


def kernel(x_drug, x_protein, edge_dd, edge_dt, edge_td, edge_pp,
           Wp_drug, bp_drug, Wp_protein, bp_protein,
           W_rel, W_self, ln_gamma, ln_beta):
    def split(e):
        e = e.astype(jnp.int32)
        return e[0].reshape(EROWS, CH), e[1].reshape(EROWS, CH)

    s_dd, d_dd = split(edge_dd)
    s_dt, d_dt = split(edge_dt)
    s_td, d_td = split(edge_td)
    s_pp, d_pp = split(edge_pp)
    zrows = jnp.zeros((125, D), jnp.float32)

    w_d0 = jnp.concatenate(
        [W_rel[0, 0].T, W_rel[0, 1].T, W_self[0, 0].T], axis=1)
    w_p0 = jnp.concatenate(
        [W_rel[0, 2].T, W_rel[0, 3].T, W_self[0, 1].T], axis=1)
    w_d1 = jnp.concatenate(
        [W_rel[1, 0].T, W_rel[1, 1].T, W_self[1, 0].T], axis=1)
    w_p1 = jnp.concatenate(
        [W_rel[1, 2].T, W_rel[1, 3].T, W_self[1, 1].T], axis=1)

    h_d, m_dd, m_dt, self_d = _projmsg(x_drug, Wp_drug.T, bp_drug[None, :],
                                       w_d0)
    h_p, m_td, m_pp, self_p = _projmsg(x_protein, Wp_protein.T,
                                       bp_protein[None, :], w_p0)
    aggd, cntd = _sc_phase_cnt(m_dd, m_td, s_dd, d_dd, s_td, d_td, zrows)
    cntd_t = cntd.T
    h_d, m_dd1, m_dt1, self_d1 = _combmsg(
        h_d, self_d, aggd, cntd_t,
        ln_gamma[0, 0][None, :], ln_beta[0, 0][None, :], w_d1)
    aggp, cntp = _sc_phase_cnt(m_dt, m_pp, s_dt, d_dt, s_pp, d_pp, zrows)
    cntp_t = cntp.T
    h_p, m_td1, m_pp1, self_p1 = _combmsg(
        h_p, self_p, aggp, cntp_t,
        ln_gamma[0, 1][None, :], ln_beta[0, 1][None, :], w_p1)
    aggd = _sc_phase_nocnt(m_dd1, m_td1, s_dd, d_dd, s_td, d_td, zrows)[0]
    h_d = _combine(h_d, self_d1, aggd, cntd_t,
                   ln_gamma[1, 0][None, :], ln_beta[1, 0][None, :])
    aggp = _sc_phase_nocnt(m_dt1, m_pp1, s_dt, d_dt, s_pp, d_pp, zrows)[0]
    h_p = _combine(h_p, self_p1, aggp, cntp_t,
                   ln_gamma[1, 1][None, :], ln_beta[1, 1][None, :])

    return jnp.concatenate([h_d, h_p], axis=0)


# speedup vs baseline: 11.0658x; 1.2154x over previous
import functools

import jax
import jax.numpy as jnp
from jax import lax
from jax.experimental import pallas as pl
from jax.experimental.pallas import tpu as pltpu
from jax.experimental.pallas import tpu_sc as plsc

ND = 10000
D = 128
DIN = 256
E = 131072
NC = 2
NS = 16
CH = 64
NBUF = 4
EROWS = E // CH
RPT = EROWS // (NC * NS)
NPT = ND // NS


def _split3(r, oa_ref, ob_ref, os_ref):
    oa_ref[...] = r[:, :D].astype(jnp.bfloat16)
    ob_ref[...] = r[:, D:2 * D].astype(jnp.bfloat16)
    os_ref[...] = r[:, 2 * D:]


def _projmsg_body(x_ref, wp_ref, bp_ref, w_ref, h_ref, oa_ref, ob_ref,
                  os_ref):
    h = jnp.maximum(
        jnp.dot(x_ref[...], wp_ref[...], preferred_element_type=jnp.float32)
        + bp_ref[...], 0.0)
    h_ref[...] = h
    _split3(jnp.dot(h, w_ref[...], preferred_element_type=jnp.float32),
            oa_ref, ob_ref, os_ref)


def _projmsg(x, wp_t, bp, w_cat):
    blk = 2000
    return pl.pallas_call(
        _projmsg_body,
        grid=(ND // blk,),
        in_specs=[pl.BlockSpec((blk, DIN), lambda i: (i, 0)),
                  pl.BlockSpec((DIN, D), lambda i: (0, 0)),
                  pl.BlockSpec((1, D), lambda i: (0, 0)),
                  pl.BlockSpec((D, 3 * D), lambda i: (0, 0))],
        out_specs=[pl.BlockSpec((blk, D), lambda i: (i, 0))] * 4,
        out_shape=[jax.ShapeDtypeStruct((ND, D), jnp.float32),
                   jax.ShapeDtypeStruct((ND, D), jnp.bfloat16),
                   jax.ShapeDtypeStruct((ND, D), jnp.bfloat16),
                   jax.ShapeDtypeStruct((ND, D), jnp.float32)] ,
    )(x, wp_t, bp, w_cat)


def _combine_val(h_ref, s_ref, agg_ref, cnt_ref, g_ref, b_ref):
    agg = (agg_ref[0].astype(jnp.float32) + agg_ref[1].astype(jnp.float32))
    cnt = jnp.sum(cnt_ref[...], axis=1, keepdims=True)
    t = jnp.maximum(s_ref[...] + agg / jnp.maximum(cnt, 1.0), 0.0) + h_ref[...]
    mu = jnp.mean(t, axis=-1, keepdims=True)
    var = jnp.mean((t - mu) ** 2, axis=-1, keepdims=True)
    return (t - mu) * lax.rsqrt(var + 1e-5) * g_ref[...] + b_ref[...]


def _combine_body(h_ref, s_ref, agg_ref, cnt_ref, g_ref, b_ref, o_ref):
    o_ref[...] = _combine_val(h_ref, s_ref, agg_ref, cnt_ref, g_ref, b_ref)


def _combmsg_body(h_ref, s_ref, agg_ref, cnt_ref, g_ref, b_ref, w_ref,
                  hn_ref, oa_ref, ob_ref, os_ref):
    y = _combine_val(h_ref, s_ref, agg_ref, cnt_ref, g_ref, b_ref)
    hn_ref[...] = y
    _split3(jnp.dot(y, w_ref[...], preferred_element_type=jnp.float32),
            oa_ref, ob_ref, os_ref)


def _combmsg(h, sf, agg, cnt, g, b, w_cat):
    blk = 2000
    return pl.pallas_call(
        _combmsg_body,
        grid=(ND // blk,),
        in_specs=[pl.BlockSpec((blk, D), lambda i: (i, 0)),
                  pl.BlockSpec((blk, D), lambda i: (i, 0)),
                  pl.BlockSpec((2, blk, D), lambda i: (0, i, 0)),
                  pl.BlockSpec((blk, NC * NS), lambda i: (i, 0)),
                  pl.BlockSpec((1, D), lambda i: (0, 0)),
                  pl.BlockSpec((1, D), lambda i: (0, 0)),
                  pl.BlockSpec((D, 3 * D), lambda i: (0, 0))],
        out_specs=[pl.BlockSpec((blk, D), lambda i: (i, 0))] * 4,
        out_shape=[jax.ShapeDtypeStruct((ND, D), jnp.float32),
                   jax.ShapeDtypeStruct((ND, D), jnp.bfloat16),
                   jax.ShapeDtypeStruct((ND, D), jnp.bfloat16),
                   jax.ShapeDtypeStruct((ND, D), jnp.float32)] ,
    )(h, sf, agg, cnt, g, b, w_cat)


def _combine(h, s, agg, cnt, g, b):
    blk = 2000
    return pl.pallas_call(
        _combine_body,
        grid=(ND // blk,),
        in_specs=[pl.BlockSpec((blk, D), lambda i: (i, 0)),
                  pl.BlockSpec((blk, D), lambda i: (i, 0)),
                  pl.BlockSpec((2, blk, D), lambda i: (0, i, 0)),
                  pl.BlockSpec((blk, NC * NS), lambda i: (i, 0)),
                  pl.BlockSpec((1, D), lambda i: (0, 0)),
                  pl.BlockSpec((1, D), lambda i: (0, 0))],
        out_specs=pl.BlockSpec((blk, D), lambda i: (i, 0)),
        out_shape=jax.ShapeDtypeStruct((ND, D), jnp.float32),
    )(h, s, agg, cnt, g, b)


def _make_sc_phase(with_cnt):
    def body(*refs):
        if with_cnt:
            (m_a, m_b, s_a, d_a, s_b, d_b, zrows, agg, cnt,
             acc, src_v, dst_v, rows0, rows1, rows2, rows3, cnt_loc,
             g0, g1, g2, g3, ss0, ss1, ss2, ss3) = refs
        else:
            (m_a, m_b, s_a, d_a, s_b, d_b, zrows, agg,
             acc, src_v, dst_v, rows0, rows1, rows2, rows3,
             g0, g1, g2, g3, ss0, ss1, ss2, ss3) = refs
        rows = (rows0, rows1, rows2, rows3)
        gsem = (g0, g1, g2, g3)
        ssem = (ss0, ss1, ss2, ss3)
        cid = lax.axis_index("c")
        sid = lax.axis_index("s")

        zero16 = jnp.zeros((16,), jnp.float32)
        one16 = jnp.ones((16,), jnp.float32)

        if with_cnt:
            def _zcnt(i, carry):
                cnt_loc[pl.ds(i * 16, 16)] = zero16
                return carry
            lax.fori_loop(0, ND // 16, _zcnt, 0)

        def _z(k, carry):
            pltpu.sync_copy(zrows, acc.at[pl.ds(sid * NPT + k * 125, 125)])
            return carry
        lax.fori_loop(0, NPT // 125, _z, 0)
        plsc.subcore_barrier()

        def do_rel(msg, s2d, d2d):
            row0 = cid * (EROWS // 2) + sid * RPT
            pltpu.sync_copy(s2d.at[pl.ds(row0, RPT)], src_v)
            pltpu.sync_copy(d2d.at[pl.ds(row0, RPT)], dst_v)

            for b in range(NBUF):
                pltpu.async_copy(msg.at[src_v.at[b]], rows[b], gsem[b])

            def _grp(o, carry):
                c0 = NBUF * o
                sdesc = []
                for b in range(NBUF):
                    c = c0 + b
                    pltpu.make_async_copy(
                        msg.at[src_v.at[c]], rows[b], gsem[b]).wait()
                    sdesc.append(pltpu.async_copy(
                        rows[b], acc.at[dst_v.at[c]], ssem[b], add=True))
                    if with_cnt:
                        for j in range(CH // 16):
                            idx = dst_v[c, pl.ds(j * 16, 16)]
                            plsc.addupdate_scatter(cnt_loc, [idx], one16)
                for b in range(NBUF):
                    sdesc[b].wait()

                    @pl.when(o < RPT // NBUF - 1)
                    def _():
                        pltpu.async_copy(
                            msg.at[src_v.at[c0 + b + NBUF]], rows[b], gsem[b])
                return carry
            lax.fori_loop(0, RPT // NBUF, _grp, 0)

        do_rel(m_a, s_a, d_a)
        do_rel(m_b, s_b, d_b)

        plsc.subcore_barrier()
        pltpu.sync_copy(acc.at[pl.ds(sid * NPT, NPT)],
                        agg.at[cid].at[pl.ds(sid * NPT, NPT)])
        if with_cnt:
            pltpu.sync_copy(cnt_loc, cnt.at[cid * NS + sid])

    out_type = [jax.ShapeDtypeStruct((NC, ND, D), jnp.bfloat16)]
    scratch = [
        pltpu.VMEM_SHARED((ND, D), jnp.bfloat16),
        pltpu.VMEM((RPT, CH), jnp.int32),
        pltpu.VMEM((RPT, CH), jnp.int32),
        pltpu.VMEM((CH, D), jnp.bfloat16),
        pltpu.VMEM((CH, D), jnp.bfloat16),
        pltpu.VMEM((CH, D), jnp.bfloat16),
        pltpu.VMEM((CH, D), jnp.bfloat16),
    ]
    if with_cnt:
        out_type.append(jax.ShapeDtypeStruct((NC * NS, ND), jnp.float32))
        scratch.append(pltpu.VMEM((ND,), jnp.float32))
    scratch += [pltpu.SemaphoreType.DMA] * 8
    return pl.kernel(
        body,
        out_type=out_type,
        mesh=plsc.VectorSubcoreMesh(core_axis_name="c", subcore_axis_name="s",
                                    num_cores=NC, num_subcores=NS),
        compiler_params=pltpu.CompilerParams(use_tc_tiling_on_sc=False,
                                             needs_layout_passes=False),
        scratch_types=scratch,
    )


_sc_phase_cnt = _make_sc_phase(True)
_sc_phase_nocnt = _make_sc_phase(False)


def kernel(x_drug, x_protein, edge_dd, edge_dt, edge_td, edge_pp,
           Wp_drug, bp_drug, Wp_protein, bp_protein,
           W_rel, W_self, ln_gamma, ln_beta):
    def split(e):
        e = e.astype(jnp.int32)
        return e[0].reshape(EROWS, CH), e[1].reshape(EROWS, CH)

    s_dd, d_dd = split(edge_dd)
    s_dt, d_dt = split(edge_dt)
    s_td, d_td = split(edge_td)
    s_pp, d_pp = split(edge_pp)
    zrows = jnp.zeros((125, D), jnp.bfloat16)

    w_d0 = jnp.concatenate(
        [W_rel[0, 0].T, W_rel[0, 1].T, W_self[0, 0].T], axis=1)
    w_p0 = jnp.concatenate(
        [W_rel[0, 2].T, W_rel[0, 3].T, W_self[0, 1].T], axis=1)
    w_d1 = jnp.concatenate(
        [W_rel[1, 0].T, W_rel[1, 1].T, W_self[1, 0].T], axis=1)
    w_p1 = jnp.concatenate(
        [W_rel[1, 2].T, W_rel[1, 3].T, W_self[1, 1].T], axis=1)

    h_d, m_dd, m_dt, self_d = _projmsg(x_drug, Wp_drug.T, bp_drug[None, :],
                                       w_d0)
    h_p, m_td, m_pp, self_p = _projmsg(x_protein, Wp_protein.T,
                                       bp_protein[None, :], w_p0)
    aggd, cntd = _sc_phase_cnt(m_dd, m_td, s_dd, d_dd, s_td, d_td, zrows)
    cntd_t = cntd.T
    h_d, m_dd1, m_dt1, self_d1 = _combmsg(
        h_d, self_d, aggd, cntd_t,
        ln_gamma[0, 0][None, :], ln_beta[0, 0][None, :], w_d1)
    aggp, cntp = _sc_phase_cnt(m_dt, m_pp, s_dt, d_dt, s_pp, d_pp, zrows)
    cntp_t = cntp.T
    h_p, m_td1, m_pp1, self_p1 = _combmsg(
        h_p, self_p, aggp, cntp_t,
        ln_gamma[0, 1][None, :], ln_beta[0, 1][None, :], w_p1)
    aggd = _sc_phase_nocnt(m_dd1, m_td1, s_dd, d_dd, s_td, d_td, zrows)[0]
    h_d = _combine(h_d, self_d1, aggd, cntd_t,
                   ln_gamma[1, 0][None, :], ln_beta[1, 0][None, :])
    aggp = _sc_phase_nocnt(m_dt1, m_pp1, s_dt, d_dt, s_pp, d_pp, zrows)[0]
    h_p = _combine(h_p, self_p1, aggp, cntp_t,
                   ln_gamma[1, 1][None, :], ln_beta[1, 1][None, :])

    return jnp.concatenate([h_d, h_p], axis=0)


# speedup vs baseline: 11.6801x; 1.0555x over previous
import functools

import jax
import jax.numpy as jnp
from jax import lax
from jax.experimental import pallas as pl
from jax.experimental.pallas import tpu as pltpu
from jax.experimental.pallas import tpu_sc as plsc

ND = 10000
D = 128
DIN = 256
E = 131072
NC = 2
NS = 16
CH = 64
NBUF = 8
EROWS = E // CH
RPT = EROWS // (NC * NS)
NPT = ND // NS


def _split3(r, oa_ref, ob_ref, os_ref):
    oa_ref[...] = r[:, :D].astype(jnp.bfloat16)
    ob_ref[...] = r[:, D:2 * D].astype(jnp.bfloat16)
    os_ref[...] = r[:, 2 * D:]


def _projmsg_body(x_ref, wp_ref, bp_ref, w_ref, h_ref, oa_ref, ob_ref,
                  os_ref):
    h = jnp.maximum(
        jnp.dot(x_ref[...], wp_ref[...], preferred_element_type=jnp.float32)
        + bp_ref[...], 0.0)
    h_ref[...] = h
    _split3(jnp.dot(h, w_ref[...], preferred_element_type=jnp.float32),
            oa_ref, ob_ref, os_ref)


def _projmsg(x, wp_t, bp, w_cat):
    blk = 2000
    return pl.pallas_call(
        _projmsg_body,
        grid=(ND // blk,),
        in_specs=[pl.BlockSpec((blk, DIN), lambda i: (i, 0)),
                  pl.BlockSpec((DIN, D), lambda i: (0, 0)),
                  pl.BlockSpec((1, D), lambda i: (0, 0)),
                  pl.BlockSpec((D, 3 * D), lambda i: (0, 0))],
        out_specs=[pl.BlockSpec((blk, D), lambda i: (i, 0))] * 4,
        out_shape=[jax.ShapeDtypeStruct((ND, D), jnp.float32),
                   jax.ShapeDtypeStruct((ND, D), jnp.bfloat16),
                   jax.ShapeDtypeStruct((ND, D), jnp.bfloat16),
                   jax.ShapeDtypeStruct((ND, D), jnp.float32)] ,
    )(x, wp_t, bp, w_cat)


def _combine_val(h_ref, s_ref, agg_ref, cnt_ref, g_ref, b_ref):
    agg = (agg_ref[0].astype(jnp.float32) + agg_ref[1].astype(jnp.float32))
    cnt = jnp.sum(cnt_ref[...], axis=1, keepdims=True)
    t = jnp.maximum(s_ref[...] + agg / jnp.maximum(cnt, 1.0), 0.0) + h_ref[...]
    mu = jnp.mean(t, axis=-1, keepdims=True)
    var = jnp.mean((t - mu) ** 2, axis=-1, keepdims=True)
    return (t - mu) * lax.rsqrt(var + 1e-5) * g_ref[...] + b_ref[...]


def _combine_body(h_ref, s_ref, agg_ref, cnt_ref, g_ref, b_ref, o_ref):
    o_ref[...] = _combine_val(h_ref, s_ref, agg_ref, cnt_ref, g_ref, b_ref)


def _combmsg_body(h_ref, s_ref, agg_ref, cnt_ref, g_ref, b_ref, w_ref,
                  hn_ref, oa_ref, ob_ref, os_ref):
    y = _combine_val(h_ref, s_ref, agg_ref, cnt_ref, g_ref, b_ref)
    hn_ref[...] = y
    _split3(jnp.dot(y, w_ref[...], preferred_element_type=jnp.float32),
            oa_ref, ob_ref, os_ref)


def _combmsg(h, sf, agg, cnt, g, b, w_cat):
    blk = 2000
    return pl.pallas_call(
        _combmsg_body,
        grid=(ND // blk,),
        in_specs=[pl.BlockSpec((blk, D), lambda i: (i, 0)),
                  pl.BlockSpec((blk, D), lambda i: (i, 0)),
                  pl.BlockSpec((2, blk, D), lambda i: (0, i, 0)),
                  pl.BlockSpec((blk, NC * NS), lambda i: (i, 0)),
                  pl.BlockSpec((1, D), lambda i: (0, 0)),
                  pl.BlockSpec((1, D), lambda i: (0, 0)),
                  pl.BlockSpec((D, 3 * D), lambda i: (0, 0))],
        out_specs=[pl.BlockSpec((blk, D), lambda i: (i, 0))] * 4,
        out_shape=[jax.ShapeDtypeStruct((ND, D), jnp.float32),
                   jax.ShapeDtypeStruct((ND, D), jnp.bfloat16),
                   jax.ShapeDtypeStruct((ND, D), jnp.bfloat16),
                   jax.ShapeDtypeStruct((ND, D), jnp.float32)] ,
    )(h, sf, agg, cnt, g, b, w_cat)


def _combine(h, s, agg, cnt, g, b):
    blk = 2000
    return pl.pallas_call(
        _combine_body,
        grid=(ND // blk,),
        in_specs=[pl.BlockSpec((blk, D), lambda i: (i, 0)),
                  pl.BlockSpec((blk, D), lambda i: (i, 0)),
                  pl.BlockSpec((2, blk, D), lambda i: (0, i, 0)),
                  pl.BlockSpec((blk, NC * NS), lambda i: (i, 0)),
                  pl.BlockSpec((1, D), lambda i: (0, 0)),
                  pl.BlockSpec((1, D), lambda i: (0, 0))],
        out_specs=pl.BlockSpec((blk, D), lambda i: (i, 0)),
        out_shape=jax.ShapeDtypeStruct((ND, D), jnp.float32),
    )(h, s, agg, cnt, g, b)


def _make_sc_phase(with_cnt):
    def body(*refs):
        if with_cnt:
            (m_a, m_b, s_a, d_a, s_b, d_b, zrows, agg, cnt,
             acc, src_v, dst_v, *rest) = refs
        else:
            (m_a, m_b, s_a, d_a, s_b, d_b, zrows, agg,
             acc, src_v, dst_v, *rest) = refs
        rows = tuple(rest[:NBUF])
        if with_cnt:
            cnt_loc = rest[NBUF]
            sems = rest[NBUF + 1:]
        else:
            sems = rest[NBUF:]
        gsem = tuple(sems[:NBUF])
        ssem = tuple(sems[NBUF:])
        cid = lax.axis_index("c")
        sid = lax.axis_index("s")

        zero16 = jnp.zeros((16,), jnp.float32)
        one16 = jnp.ones((16,), jnp.float32)

        if with_cnt:
            def _zcnt(i, carry):
                cnt_loc[pl.ds(i * 16, 16)] = zero16
                return carry
            lax.fori_loop(0, ND // 16, _zcnt, 0)

        def _z(k, carry):
            pltpu.sync_copy(zrows, acc.at[pl.ds(sid * NPT + k * 125, 125)])
            return carry
        lax.fori_loop(0, NPT // 125, _z, 0)
        plsc.subcore_barrier()

        def do_rel(msg, s2d, d2d):
            row0 = cid * (EROWS // 2) + sid * RPT
            pltpu.sync_copy(s2d.at[pl.ds(row0, RPT)], src_v)
            pltpu.sync_copy(d2d.at[pl.ds(row0, RPT)], dst_v)

            for b in range(NBUF):
                pltpu.async_copy(msg.at[src_v.at[b]], rows[b], gsem[b])

            def _grp(o, carry):
                c0 = NBUF * o
                sdesc = []
                for b in range(NBUF):
                    c = c0 + b
                    pltpu.make_async_copy(
                        msg.at[src_v.at[c]], rows[b], gsem[b]).wait()
                    sdesc.append(pltpu.async_copy(
                        rows[b], acc.at[dst_v.at[c]], ssem[b], add=True))
                    if with_cnt:
                        for j in range(CH // 16):
                            idx = dst_v[c, pl.ds(j * 16, 16)]
                            plsc.addupdate_scatter(cnt_loc, [idx], one16)
                for b in range(NBUF):
                    sdesc[b].wait()

                    @pl.when(o < RPT // NBUF - 1)
                    def _():
                        pltpu.async_copy(
                            msg.at[src_v.at[c0 + b + NBUF]], rows[b], gsem[b])
                return carry
            lax.fori_loop(0, RPT // NBUF, _grp, 0)

        do_rel(m_a, s_a, d_a)
        do_rel(m_b, s_b, d_b)

        plsc.subcore_barrier()
        pltpu.sync_copy(acc.at[pl.ds(sid * NPT, NPT)],
                        agg.at[cid].at[pl.ds(sid * NPT, NPT)])
        if with_cnt:
            pltpu.sync_copy(cnt_loc, cnt.at[cid * NS + sid])

    out_type = [jax.ShapeDtypeStruct((NC, ND, D), jnp.bfloat16)]
    scratch = [
        pltpu.VMEM_SHARED((ND, D), jnp.bfloat16),
        pltpu.VMEM((RPT, CH), jnp.int32),
        pltpu.VMEM((RPT, CH), jnp.int32),
    ] + [pltpu.VMEM((CH, D), jnp.bfloat16)] * NBUF
    if with_cnt:
        out_type.append(jax.ShapeDtypeStruct((NC * NS, ND), jnp.float32))
        scratch.append(pltpu.VMEM((ND,), jnp.float32))
    scratch += [pltpu.SemaphoreType.DMA] * (2 * NBUF)
    return pl.kernel(
        body,
        out_type=out_type,
        mesh=plsc.VectorSubcoreMesh(core_axis_name="c", subcore_axis_name="s",
                                    num_cores=NC, num_subcores=NS),
        compiler_params=pltpu.CompilerParams(use_tc_tiling_on_sc=False,
                                             needs_layout_passes=False),
        scratch_types=scratch,
    )


_sc_phase_cnt = _make_sc_phase(True)
_sc_phase_nocnt = _make_sc_phase(False)


def kernel(x_drug, x_protein, edge_dd, edge_dt, edge_td, edge_pp,
           Wp_drug, bp_drug, Wp_protein, bp_protein,
           W_rel, W_self, ln_gamma, ln_beta):
    def split(e):
        e = e.astype(jnp.int32)
        return e[0].reshape(EROWS, CH), e[1].reshape(EROWS, CH)

    s_dd, d_dd = split(edge_dd)
    s_dt, d_dt = split(edge_dt)
    s_td, d_td = split(edge_td)
    s_pp, d_pp = split(edge_pp)
    zrows = jnp.zeros((125, D), jnp.bfloat16)

    w_d0 = jnp.concatenate(
        [W_rel[0, 0].T, W_rel[0, 1].T, W_self[0, 0].T], axis=1)
    w_p0 = jnp.concatenate(
        [W_rel[0, 2].T, W_rel[0, 3].T, W_self[0, 1].T], axis=1)
    w_d1 = jnp.concatenate(
        [W_rel[1, 0].T, W_rel[1, 1].T, W_self[1, 0].T], axis=1)
    w_p1 = jnp.concatenate(
        [W_rel[1, 2].T, W_rel[1, 3].T, W_self[1, 1].T], axis=1)

    h_d, m_dd, m_dt, self_d = _projmsg(x_drug, Wp_drug.T, bp_drug[None, :],
                                       w_d0)
    h_p, m_td, m_pp, self_p = _projmsg(x_protein, Wp_protein.T,
                                       bp_protein[None, :], w_p0)
    aggd, cntd = _sc_phase_cnt(m_dd, m_td, s_dd, d_dd, s_td, d_td, zrows)
    cntd_t = cntd.T
    h_d, m_dd1, m_dt1, self_d1 = _combmsg(
        h_d, self_d, aggd, cntd_t,
        ln_gamma[0, 0][None, :], ln_beta[0, 0][None, :], w_d1)
    aggp, cntp = _sc_phase_cnt(m_dt, m_pp, s_dt, d_dt, s_pp, d_pp, zrows)
    cntp_t = cntp.T
    h_p, m_td1, m_pp1, self_p1 = _combmsg(
        h_p, self_p, aggp, cntp_t,
        ln_gamma[0, 1][None, :], ln_beta[0, 1][None, :], w_p1)
    aggd = _sc_phase_nocnt(m_dd1, m_td1, s_dd, d_dd, s_td, d_td, zrows)[0]
    h_d = _combine(h_d, self_d1, aggd, cntd_t,
                   ln_gamma[1, 0][None, :], ln_beta[1, 0][None, :])
    aggp = _sc_phase_nocnt(m_dt1, m_pp1, s_dt, d_dt, s_pp, d_pp, zrows)[0]
    h_p = _combine(h_p, self_p1, aggp, cntp_t,
                   ln_gamma[1, 1][None, :], ln_beta[1, 1][None, :])

    return jnp.concatenate([h_d, h_p], axis=0)
